# Initial kernel scaffold; baseline (speedup 1.0000x reference)
import jax
import jax.numpy as jnp
from jax.experimental import pallas as pl


def kernel(features, lap_indices, lap_values):
    raise NotImplementedError("write your pallas kernel here")


# speedup vs baseline: 4.5683x; 4.5683x over previous
import functools

import jax
import jax.numpy as jnp
from jax import lax
from jax.experimental import pallas as pl
from jax.experimental.pallas import tpu as pltpu
from jax.experimental.pallas import tpu_sc as plsc

N = 10000
E = 320000
D = 128

NC = 2
NS = 16
NW = NC * NS

C = 128
NCH = -(-E // (NW * C))
EP = NW * NCH * C

LPR = D // 16
RPT = 624
TAIL = N - NS * RPT

_mesh = plsc.VectorSubcoreMesh(core_axis_name="c", subcore_axis_name="s")


@functools.partial(
    pl.kernel,
    mesh=_mesh,
    out_type=jax.ShapeDtypeStruct((NC, N, D), jnp.float32),
    scratch_types=[
        pltpu.VMEM((NCH, C), jnp.int32),
        pltpu.VMEM((NCH, C), jnp.int32),
        pltpu.VMEM((NCH, C), jnp.float32),
        pltpu.VMEM((C, D), jnp.float32),
        pltpu.VMEM_SHARED((N, D), jnp.float32),
        pltpu.SemaphoreType.DMA,
    ],
)
def _scatter_kernel(feat, col3, row3, val3, out, colbuf, rowbuf, valbuf,
                    rbuf, acc, sem):
    c = lax.axis_index("c")
    s = lax.axis_index("s")
    wid = s * NC + c
    zero16 = jnp.zeros((16,), jnp.float32)

    def z_body(r, carry):
        for k in range(LPR):
            rbuf[r, pl.ds(k * 16, 16)] = zero16
        return carry
    lax.fori_loop(0, C, z_body, 0)
    base = s * RPT
    rem = RPT % C
    for t in range(RPT // C):
        pltpu.sync_copy(rbuf, acc.at[pl.ds(base + t * C, C)])
    pltpu.sync_copy(rbuf.at[pl.ds(0, rem)],
                    acc.at[pl.ds(base + (RPT // C) * C, rem)])

    @pl.when(s == NS - 1)
    def _zero_tail():
        pltpu.sync_copy(rbuf.at[pl.ds(0, TAIL)], acc.at[pl.ds(NS * RPT, TAIL)])
    plsc.subcore_barrier()

    pltpu.sync_copy(col3.at[wid], colbuf)
    pltpu.sync_copy(row3.at[wid], rowbuf)
    pltpu.sync_copy(val3.at[wid], valbuf)

    def chunk_body(j, carry):
        pltpu.async_copy(feat.at[colbuf.at[j]], rbuf, sem).wait()

        def mul_body(g, carry2):
            vv = valbuf[j, pl.ds(g * 16, 16)]
            for u in range(16):
                v = vv[u]
                e = g * 16 + u
                for k in range(LPR):
                    rbuf[e, pl.ds(k * 16, 16)] = rbuf[e, pl.ds(k * 16, 16)] * v
            return carry2
        lax.fori_loop(0, C // 16, mul_body, 0)

        pltpu.sync_copy(rbuf, acc.at[rowbuf.at[j]], add=True)
        return carry
    lax.fori_loop(0, NCH, chunk_body, 0)
    plsc.subcore_barrier()

    nfull = RPT // C
    for t in range(nfull + 1):
        sz = C if t < nfull else rem
        r0 = base + t * C
        pltpu.sync_copy(acc.at[pl.ds(r0, sz)], rbuf.at[pl.ds(0, sz)])
        pltpu.sync_copy(rbuf.at[pl.ds(0, sz)], out.at[c, pl.ds(r0, sz)])

    @pl.when(s == NS - 1)
    def _write_tail():
        pltpu.sync_copy(acc.at[pl.ds(NS * RPT, TAIL)], rbuf.at[pl.ds(0, TAIL)])
        pltpu.sync_copy(rbuf.at[pl.ds(0, TAIL)], out.at[c, pl.ds(NS * RPT, TAIL)])


def _combine_body(p0, p1, f, o):
    o[...] = p0[0] + p1[0] + f[...]


_BLK = 1000


def _combine(partials, features):
    return pl.pallas_call(
        _combine_body,
        grid=(N // _BLK,),
        in_specs=[
            pl.BlockSpec((1, _BLK, D), lambda i: (0, i, 0)),
            pl.BlockSpec((1, _BLK, D), lambda i: (1, i, 0)),
            pl.BlockSpec((_BLK, D), lambda i: (i, 0)),
        ],
        out_specs=pl.BlockSpec((_BLK, D), lambda i: (i, 0)),
        out_shape=jax.ShapeDtypeStruct((N, D), jnp.float32),
    )(partials, partials, features)


def kernel(features, lap_indices, lap_values):
    pad = EP - E
    row = jnp.pad(lap_indices[0], (0, pad)).reshape(NW, NCH, C)
    col = jnp.pad(lap_indices[1], (0, pad)).reshape(NW, NCH, C)
    val = jnp.pad(lap_values, (0, pad)).reshape(NW, NCH, C)
    partials = _scatter_kernel(features, col, row, val)
    return _combine(partials, features)
